# baseline (speedup 1.0000x reference)
import functools

import jax
import jax.numpy as jnp
from jax import lax
from jax.experimental import pallas as pl
from jax.experimental.pallas import tpu as pltpu
from jax.experimental.pallas import tpu_sc as plsc

HID = 768
EPS = 1e-12

_NUM_CORES = 2
_NUM_SUBCORES = 16
_NW = _NUM_CORES * _NUM_SUBCORES

_CHUNK = 64


def _sc_gather(table, idx2d):
    n_chunks = idx2d.shape[0]
    chunks_per_w = n_chunks // _NW
    n_tokens = n_chunks * _CHUNK
    mesh = plsc.VectorSubcoreMesh(core_axis_name="c", subcore_axis_name="s")

    @functools.partial(
        pl.kernel,
        mesh=mesh,
        out_type=jax.ShapeDtypeStruct((n_tokens, HID), jnp.float32),
        scratch_types=[
            pltpu.VMEM((chunks_per_w, _CHUNK), jnp.int32),
            pltpu.VMEM((2, _CHUNK, HID), jnp.float32),
            pltpu.SemaphoreType.DMA,
            pltpu.SemaphoreType.DMA,
        ],
    )
    def gather_kernel(table_hbm, idx_hbm, out_hbm, idx_v, rows_v, gsem, osem):
        wid = lax.axis_index("s") * _NUM_CORES + lax.axis_index("c")
        chunk0 = wid * chunks_per_w
        base = chunk0 * _CHUNK
        pltpu.sync_copy(idx_hbm.at[pl.ds(chunk0, chunks_per_w)], idx_v)

        def start_gather(j):
            return pltpu.async_copy(
                table_hbm.at[idx_v.at[j]], rows_v.at[j % 2], gsem)

        def start_out(j):
            return pltpu.async_copy(
                rows_v.at[j % 2],
                out_hbm.at[pl.ds(base + j * _CHUNK, _CHUNK)], osem)

        outs = [None] * chunks_per_w
        g = start_gather(0)
        for j in range(chunks_per_w):
            g.wait()
            outs[j] = start_out(j)
            if j + 1 < chunks_per_w:
                if j >= 1:
                    outs[j - 1].wait()
                g = start_gather(j + 1)
        if chunks_per_w >= 2:
            outs[chunks_per_w - 2].wait()
        outs[chunks_per_w - 1].wait()

    return gather_kernel(table, idx2d)


def _ln_body(x_ref, pos_ref, type_ref, gamma_ref, beta_ref, o_ref):
    x = x_ref[0] + pos_ref[...] + type_ref[...]
    mean = jnp.mean(x, axis=-1, keepdims=True)
    xc = x - mean
    var = jnp.mean(xc * xc, axis=-1, keepdims=True)
    o_ref[0] = xc * lax.rsqrt(var + EPS) * gamma_ref[...] + beta_ref[...]


def _ln(x, pos_emb, type_row, gamma, beta):
    B, S, _ = x.shape
    BS = 512
    grid = (B, S // BS)
    return pl.pallas_call(
        _ln_body,
        grid=grid,
        in_specs=[
            pl.BlockSpec((1, BS, HID), lambda b, j: (b, j, 0)),
            pl.BlockSpec((BS, HID), lambda b, j: (j, 0)),
            pl.BlockSpec((1, HID), lambda b, j: (0, 0)),
            pl.BlockSpec((1, HID), lambda b, j: (0, 0)),
            pl.BlockSpec((1, HID), lambda b, j: (0, 0)),
        ],
        out_specs=pl.BlockSpec((1, BS, HID), lambda b, j: (b, j, 0)),
        out_shape=jax.ShapeDtypeStruct((B, S, HID), jnp.float32),
    )(x, pos_emb, type_row, gamma, beta)


_K_SLABS = 2


def kernel(input_ids, word_emb, pos_emb, type_emb, ln_gamma, ln_beta):
    B, S = input_ids.shape
    n = B * S
    ids = input_ids.reshape(-1).astype(jnp.int32)
    b_per_slab = B // _K_SLABS
    slab = n // _K_SLABS
    type_row = type_emb[0:1]
    gamma = ln_gamma.reshape(1, HID)
    beta = ln_beta.reshape(1, HID)
    outs = []
    for k in range(_K_SLABS):
        idx2d = lax.dynamic_slice_in_dim(ids, k * slab, slab).reshape(
            slab // _CHUNK, _CHUNK)
        g = _sc_gather(word_emb, idx2d).reshape(b_per_slab, S, HID)
        outs.append(_ln(g, pos_emb, type_row, gamma, beta))
    return jnp.concatenate(outs, axis=0)


# speedup vs baseline: 1.3231x; 1.3231x over previous
import functools

import jax
import jax.numpy as jnp
from jax import lax
from jax.experimental import pallas as pl
from jax.experimental.pallas import tpu as pltpu
from jax.experimental.pallas import tpu_sc as plsc

HID = 768
EPS = 1e-12

_NUM_CORES = 2
_NUM_SUBCORES = 16
_NW = _NUM_CORES * _NUM_SUBCORES

_CHUNK = 64


def _sc_gather(table, idx2d):
    n_chunks = idx2d.shape[0]
    chunks_per_w = n_chunks // _NW
    n_tokens = n_chunks * _CHUNK
    mesh = plsc.VectorSubcoreMesh(core_axis_name="c", subcore_axis_name="s")

    @functools.partial(
        pl.kernel,
        mesh=mesh,
        out_type=jax.ShapeDtypeStruct((n_tokens, HID), jnp.float32),
        scratch_types=[
            pltpu.VMEM((chunks_per_w, _CHUNK), jnp.int32),
            pltpu.VMEM((2, _CHUNK, HID), jnp.float32),
            pltpu.SemaphoreType.DMA,
            pltpu.SemaphoreType.DMA,
        ],
    )
    def gather_kernel(table_hbm, idx_hbm, out_hbm, idx_v, rows_v, gsem, osem):
        wid = lax.axis_index("s") * _NUM_CORES + lax.axis_index("c")
        chunk0 = wid * chunks_per_w
        base = chunk0 * _CHUNK
        pltpu.sync_copy(idx_hbm.at[pl.ds(chunk0, chunks_per_w)], idx_v)

        def start_gather(j):
            return pltpu.async_copy(
                table_hbm.at[idx_v.at[j]], rows_v.at[j % 2], gsem)

        def start_out(j):
            return pltpu.async_copy(
                rows_v.at[j % 2],
                out_hbm.at[pl.ds(base + j * _CHUNK, _CHUNK)], osem)

        outs = [None] * chunks_per_w
        g = start_gather(0)
        for j in range(chunks_per_w):
            g.wait()
            outs[j] = start_out(j)
            if j + 1 < chunks_per_w:
                if j >= 1:
                    outs[j - 1].wait()
                g = start_gather(j + 1)
        if chunks_per_w >= 2:
            outs[chunks_per_w - 2].wait()
        outs[chunks_per_w - 1].wait()

    return gather_kernel(table, idx2d)


def _ln_body(x_ref, pos_ref, type_ref, gamma_ref, beta_ref, o_ref):
    x = x_ref[0] + pos_ref[...] + type_ref[...]
    mean = jnp.mean(x, axis=-1, keepdims=True)
    xc = x - mean
    var = jnp.mean(xc * xc, axis=-1, keepdims=True)
    o_ref[0] = xc * lax.rsqrt(var + EPS) * gamma_ref[...] + beta_ref[...]


def _ln(x, pos_emb, type_row, gamma, beta):
    B, S, _ = x.shape
    BS = 512
    grid = (S // BS, B)
    return pl.pallas_call(
        _ln_body,
        grid=grid,
        in_specs=[
            pl.BlockSpec((1, BS, HID), lambda j, b: (b, j, 0)),
            pl.BlockSpec((BS, HID), lambda j, b: (j, 0)),
            pl.BlockSpec((1, HID), lambda j, b: (0, 0)),
            pl.BlockSpec((1, HID), lambda j, b: (0, 0)),
            pl.BlockSpec((1, HID), lambda j, b: (0, 0)),
        ],
        out_specs=pl.BlockSpec((1, BS, HID), lambda j, b: (b, j, 0)),
        out_shape=jax.ShapeDtypeStruct((B, S, HID), jnp.float32),
    )(x, pos_emb, type_row, gamma, beta)


def kernel(input_ids, word_emb, pos_emb, type_emb, ln_gamma, ln_beta):
    B, S = input_ids.shape
    n = B * S
    ids = input_ids.reshape(-1).astype(jnp.int32)
    idx2d = ids.reshape(n // _CHUNK, _CHUNK)
    gathered = _sc_gather(word_emb, idx2d)
    x = gathered.reshape(B, S, HID)
    return _ln(x, pos_emb, type_emb[0:1],
               ln_gamma.reshape(1, HID), ln_beta.reshape(1, HID))


# speedup vs baseline: 1.3747x; 1.0390x over previous
import functools

import jax
import jax.numpy as jnp
from jax import lax
from jax.experimental import pallas as pl
from jax.experimental.pallas import tpu as pltpu
from jax.experimental.pallas import tpu_sc as plsc

HID = 768
EPS = 1e-12

_NUM_CORES = 2
_NUM_SUBCORES = 16
_NW = _NUM_CORES * _NUM_SUBCORES

_CHUNK = 64


def _sc_gather(table, idx2d):
    n_chunks = idx2d.shape[0]
    chunks_per_w = n_chunks // _NW
    n_tokens = n_chunks * _CHUNK
    mesh = plsc.VectorSubcoreMesh(core_axis_name="c", subcore_axis_name="s")

    @functools.partial(
        pl.kernel,
        mesh=mesh,
        out_type=jax.ShapeDtypeStruct((n_tokens, HID), jnp.float32),
        scratch_types=[
            pltpu.VMEM((chunks_per_w, _CHUNK), jnp.int32),
            pltpu.VMEM((2, _CHUNK, HID), jnp.float32),
            pltpu.SemaphoreType.DMA,
            pltpu.SemaphoreType.DMA,
        ],
    )
    def gather_kernel(table_hbm, idx_hbm, out_hbm, idx_v, rows_v, gsem, osem):
        wid = lax.axis_index("s") * _NUM_CORES + lax.axis_index("c")
        chunk0 = wid * chunks_per_w
        base = chunk0 * _CHUNK
        pltpu.sync_copy(idx_hbm.at[pl.ds(chunk0, chunks_per_w)], idx_v)

        def start_gather(j):
            return pltpu.async_copy(
                table_hbm.at[idx_v.at[j]], rows_v.at[j % 2], gsem)

        def start_out(j):
            return pltpu.async_copy(
                rows_v.at[j % 2],
                out_hbm.at[pl.ds(base + j * _CHUNK, _CHUNK)], osem)

        outs = [None] * chunks_per_w
        g = start_gather(0)
        for j in range(chunks_per_w):
            g.wait()
            outs[j] = start_out(j)
            if j + 1 < chunks_per_w:
                if j >= 1:
                    outs[j - 1].wait()
                g = start_gather(j + 1)
        if chunks_per_w >= 2:
            outs[chunks_per_w - 2].wait()
        outs[chunks_per_w - 1].wait()

    return gather_kernel(table, idx2d)


def _ln_body(x_ref, pos_ref, gamma_ref, beta_ref, o_ref):
    x = x_ref[0] + pos_ref[...]
    mean = jnp.mean(x, axis=-1, keepdims=True)
    msq = jnp.mean(x * x, axis=-1, keepdims=True)
    var = msq - mean * mean
    rstd = lax.rsqrt(var + EPS)
    o_ref[0] = (x - mean) * (rstd * gamma_ref[...]) + beta_ref[...]


def _ln(x, posplus, gamma, beta):
    B, S, _ = x.shape
    BS = 1024
    grid = (S // BS, B)
    return pl.pallas_call(
        _ln_body,
        grid=grid,
        in_specs=[
            pl.BlockSpec((1, BS, HID), lambda j, b: (b, j, 0)),
            pl.BlockSpec((BS, HID), lambda j, b: (j, 0)),
            pl.BlockSpec((1, HID), lambda j, b: (0, 0)),
            pl.BlockSpec((1, HID), lambda j, b: (0, 0)),
        ],
        out_specs=pl.BlockSpec((1, BS, HID), lambda j, b: (b, j, 0)),
        out_shape=jax.ShapeDtypeStruct((B, S, HID), jnp.float32),
    )(x, posplus, gamma, beta)


def kernel(input_ids, word_emb, pos_emb, type_emb, ln_gamma, ln_beta):
    B, S = input_ids.shape
    n = B * S
    ids = input_ids.reshape(-1).astype(jnp.int32)
    idx2d = ids.reshape(n // _CHUNK, _CHUNK)
    gathered = _sc_gather(word_emb, idx2d)
    x = gathered.reshape(B, S, HID)
    posplus = pos_emb + type_emb[0:1]
    return _ln(x, posplus,
               ln_gamma.reshape(1, HID), ln_beta.reshape(1, HID))
